# baseline (speedup 1.0000x reference)
import functools

import jax
import jax.numpy as jnp
from jax import lax
from jax.experimental import pallas as pl
from jax.experimental.pallas import tpu as pltpu
from jax.experimental.pallas import tpu_sc as plsc

_CHUNK = 128
_NBUF = 10


def _make_gather(num_idx: int, dim: int, nc: int, ns: int):
  nw = nc * ns
  b_per_w = num_idx // nw
  chunks = b_per_w // _CHUNK
  groups = chunks // _NBUF - 1

  mesh = plsc.VectorSubcoreMesh(core_axis_name="c", subcore_axis_name="s")

  @functools.partial(
      pl.kernel,
      out_type=jax.ShapeDtypeStruct((num_idx, dim), jnp.float32),
      mesh=mesh,
      scratch_types=[
          pltpu.VMEM((chunks, _CHUNK), jnp.int32),
          pltpu.VMEM((_NBUF, _CHUNK, dim), jnp.float32),
      ] + [pltpu.SemaphoreType.DMA] * _NBUF,
      compiler_params=pltpu.CompilerParams(use_tc_tiling_on_sc=False),
  )
  def gather_kernel(tok_hbm, table_hbm, out_hbm, idx_v, rows_v, *sems):
    wid = lax.axis_index("s") * nc + lax.axis_index("c")
    base = wid * b_per_w

    pltpu.sync_copy(tok_hbm.at[wid], idx_v)

    for b in range(_NBUF):
      pltpu.async_copy(table_hbm.at[idx_v.at[b]], rows_v.at[b], sems[b])

    @pl.loop(0, groups)
    def _(g):
      for b in range(_NBUF):
        j = g * _NBUF + b
        pltpu.make_async_copy(table_hbm.at[idx_v.at[j]], rows_v.at[b],
                              sems[b]).wait()
        pltpu.sync_copy(rows_v.at[b],
                        out_hbm.at[pl.ds(base + j * _CHUNK, _CHUNK)])
        pltpu.async_copy(table_hbm.at[idx_v.at[j + _NBUF]], rows_v.at[b],
                         sems[b])

    for b in range(_NBUF):
      j = groups * _NBUF + b
      pltpu.make_async_copy(table_hbm.at[idx_v.at[j]], rows_v.at[b],
                            sems[b]).wait()
      pltpu.sync_copy(rows_v.at[b],
                      out_hbm.at[pl.ds(base + j * _CHUNK, _CHUNK)])

  return gather_kernel


def kernel(token_ids, weight):
  info = plsc.get_sparse_core_info()
  num_idx = token_ids.size
  dim = weight.shape[1]
  nw = info.num_cores * info.num_subcores
  tok = token_ids.astype(jnp.int32).reshape(
      nw, num_idx // (nw * _CHUNK), _CHUNK)
  out = _make_gather(num_idx, dim, info.num_cores, info.num_subcores)(
      tok, weight)
  return out.reshape(token_ids.shape + (dim,))


# speedup vs baseline: 1.0040x; 1.0040x over previous
import functools

import jax
import jax.numpy as jnp
from jax import lax
from jax.experimental import pallas as pl
from jax.experimental.pallas import tpu as pltpu
from jax.experimental.pallas import tpu_sc as plsc

_CHUNK = 256
_NBUF = 5


def _make_gather(num_idx: int, dim: int, nc: int, ns: int):
  nw = nc * ns
  b_per_w = num_idx // nw
  chunks = b_per_w // _CHUNK
  groups = chunks // _NBUF - 1

  mesh = plsc.VectorSubcoreMesh(core_axis_name="c", subcore_axis_name="s")

  @functools.partial(
      pl.kernel,
      out_type=jax.ShapeDtypeStruct((num_idx, dim), jnp.float32),
      mesh=mesh,
      scratch_types=[
          pltpu.VMEM((chunks, _CHUNK), jnp.int32),
          pltpu.VMEM((_NBUF, _CHUNK, dim), jnp.float32),
      ] + [pltpu.SemaphoreType.DMA] * _NBUF,
      compiler_params=pltpu.CompilerParams(use_tc_tiling_on_sc=False),
  )
  def gather_kernel(tok_hbm, table_hbm, out_hbm, idx_v, rows_v, *sems):
    wid = lax.axis_index("s") * nc + lax.axis_index("c")
    base = wid * b_per_w

    pltpu.sync_copy(tok_hbm.at[wid], idx_v)

    for b in range(_NBUF):
      pltpu.async_copy(table_hbm.at[idx_v.at[b]], rows_v.at[b], sems[b])

    @pl.loop(0, groups)
    def _(g):
      for b in range(_NBUF):
        j = g * _NBUF + b
        pltpu.make_async_copy(table_hbm.at[idx_v.at[j]], rows_v.at[b],
                              sems[b]).wait()
        pltpu.sync_copy(rows_v.at[b],
                        out_hbm.at[pl.ds(base + j * _CHUNK, _CHUNK)])
        pltpu.async_copy(table_hbm.at[idx_v.at[j + _NBUF]], rows_v.at[b],
                         sems[b])

    for b in range(_NBUF):
      j = groups * _NBUF + b
      pltpu.make_async_copy(table_hbm.at[idx_v.at[j]], rows_v.at[b],
                            sems[b]).wait()
      pltpu.sync_copy(rows_v.at[b],
                      out_hbm.at[pl.ds(base + j * _CHUNK, _CHUNK)])

  return gather_kernel


def kernel(token_ids, weight):
  info = plsc.get_sparse_core_info()
  num_idx = token_ids.size
  dim = weight.shape[1]
  nw = info.num_cores * info.num_subcores
  tok = token_ids.astype(jnp.int32).reshape(
      nw, num_idx // (nw * _CHUNK), _CHUNK)
  out = _make_gather(num_idx, dim, info.num_cores, info.num_subcores)(
      tok, weight)
  return out.reshape(token_ids.shape + (dim,))
